# baseline (speedup 1.0000x reference)
import dataclasses
import functools

import jax
import jax.numpy as jnp
from jax import lax
from jax.experimental import pallas as pl
from jax.experimental.pallas import tpu as pltpu
from jax.experimental.pallas import tpu_sc as plsc

N = 10000
E = 160000
EP = 163840
NW = 32
EPW = EP // NW
NPT = 320
NPAD = NW * NPT
CL = 5120
CL3 = 3 * CL
NCH = EP // CL

_MESH = plsc.VectorSubcoreMesh(core_axis_name="c", subcore_axis_name="s")

_SC_PARAMS = pltpu.CompilerParams()
if "needs_layout_passes" in pltpu.CompilerParams.__dataclass_fields__:
    _SC_PARAMS = dataclasses.replace(_SC_PARAMS, needs_layout_passes=False)


def _wid():
    return lax.axis_index("s") * 2 + lax.axis_index("c")


def _deg_partials(col, ew):
    @functools.partial(
        pl.kernel,
        out_type=jax.ShapeDtypeStruct((NW * N,), jnp.float32),
        mesh=_MESH,
        compiler_params=_SC_PARAMS,
        scratch_types=[
            pltpu.VMEM((EPW + 16,), jnp.int32),
            pltpu.VMEM((EPW + 16,), jnp.float32),
            pltpu.VMEM((N,), jnp.float32),
        ],
    )
    def k(col_hbm, ew_hbm, out_hbm, colv, eww, hist):
        w = _wid()
        base = w * EPW
        pltpu.sync_copy(col_hbm.at[pl.ds(base, EPW)], colv.at[pl.ds(0, EPW)])
        pltpu.sync_copy(ew_hbm.at[pl.ds(base, EPW)], eww.at[pl.ds(0, EPW)])

        @pl.loop(0, N // 16)
        def _(i):
            hist[pl.ds(i * 16, 16)] = jnp.zeros((16,), jnp.float32)

        lane0 = lax.iota(jnp.int32, 16) == 0

        @pl.loop(0, EPW)
        def _(e):
            plsc.addupdate_scatter(hist, [colv[pl.ds(e, 16)]],
                                   eww[pl.ds(e, 16)], mask=lane0)

        pltpu.sync_copy(hist, out_hbm.at[pl.ds(w * N, N)])

    return k(col, ew)


def _norm_kernel(row, col, ew, dinv):
    @functools.partial(
        pl.kernel,
        out_type=jax.ShapeDtypeStruct((EP,), jnp.float32),
        mesh=_MESH,
        compiler_params=_SC_PARAMS,
        scratch_types=[
            pltpu.VMEM((EPW,), jnp.int32),
            pltpu.VMEM((EPW,), jnp.int32),
            pltpu.VMEM((EPW,), jnp.float32),
            pltpu.VMEM((EPW,), jnp.float32),
            pltpu.VMEM((N,), jnp.float32),
        ],
    )
    def k(row_hbm, col_hbm, ew_hbm, dinv_hbm, out_hbm, rv, cv, wv, nv, dv):
        w = _wid()
        base = w * EPW
        pltpu.sync_copy(row_hbm.at[pl.ds(base, EPW)], rv)
        pltpu.sync_copy(col_hbm.at[pl.ds(base, EPW)], cv)
        pltpu.sync_copy(ew_hbm.at[pl.ds(base, EPW)], wv)
        pltpu.sync_copy(dinv_hbm, dv)

        @pl.loop(0, EPW // 16)
        def _(g):
            sl = pl.ds(g * 16, 16)
            dr = plsc.load_gather(dv, [rv[sl]])
            dc = plsc.load_gather(dv, [cv[sl]])
            nv[sl] = dr * wv[sl] * dc

        pltpu.sync_copy(nv, out_hbm.at[pl.ds(base, EPW)])

    return k(row, col, ew, dinv)


def _agg_kernel(xw, row, col, nrm, d, d_tab, g_rows):
    @functools.partial(
        pl.kernel,
        out_type=jax.ShapeDtypeStruct((NPAD * d,), jnp.float32),
        mesh=_MESH,
        compiler_params=_SC_PARAMS,
        scratch_types=[
            pltpu.VMEM((CL,), jnp.int32),
            pltpu.VMEM((CL,), jnp.int32),
            pltpu.VMEM((CL,), jnp.float32),
            pltpu.VMEM((CL,), jnp.int32),
            pltpu.VMEM((CL + 16,), jnp.int32),
            pltpu.VMEM((CL + 16,), jnp.float32),
            pltpu.VMEM((g_rows, d_tab), jnp.float32),
            pltpu.VMEM((NPT * d,), jnp.float32),
        ],
    )
    def k(xw_hbm, row_hbm, col_hbm, nrm_hbm, out_hbm, rs, cs, ns, rb, lb, nb, gb, acc):
        w = _wid()
        nbase = w * NPT

        @pl.loop(0, NPT * d // 16)
        def _(i):
            acc[pl.ds(i * 16, 16)] = jnp.zeros((16,), jnp.float32)

        @pl.loop(0, CL // 16)
        def _(i):
            rb[pl.ds(i * 16, 16)] = jnp.zeros((16,), jnp.int32)

        @pl.loop(0, NCH)
        def _(ch):
            eoff = ch * CL
            pltpu.sync_copy(row_hbm.at[pl.ds(eoff, CL)], rs)
            pltpu.sync_copy(col_hbm.at[pl.ds(eoff, CL)], cs)
            pltpu.sync_copy(nrm_hbm.at[pl.ds(eoff, CL)], ns)

            def grp(g, pos):
                sl = pl.ds(g * 16, 16)
                r16 = rs[sl]
                c16 = cs[sl]
                n16 = ns[sl]
                lc = c16 - nbase
                m = (lc >= 0) & (lc < NPT) & (c16 < N)
                plsc.store_compressed(rb.at[pl.ds(pos, 16)], r16, mask=m)
                plsc.store_compressed(lb.at[pl.ds(pos, 16)], lc, mask=m)
                plsc.store_compressed(nb.at[pl.ds(pos, 16)], n16, mask=m)
                return pos + jnp.sum(m.astype(jnp.int32))

            n_in = lax.fori_loop(0, CL // 16, grp, 0)
            nblk = (n_in + g_rows - 1) // g_rows

            def blk(b, _):
                pltpu.sync_copy(xw_hbm.at[rb.at[pl.ds(b * g_rows, g_rows)]],
                                gb)
                nleft = jnp.minimum(n_in - b * g_rows, g_rows)

                def edge(e, _):
                    pos = b * g_rows + e
                    lc_s = lb[pl.ds(pos, 16)][0]
                    nmv = jnp.full((16,), nb[pl.ds(pos, 16)][0], jnp.float32)
                    for f in range(d // 16):
                        g16 = gb[e, pl.ds(f * 16, 16)]
                        plsc.addupdate(acc.at[pl.ds(lc_s * d + f * 16, 16)],
                                       nmv * g16)
                    return 0

                lax.fori_loop(0, nleft, edge, 0)
                return 0

            lax.fori_loop(0, nblk, blk, 0)

        pltpu.sync_copy(acc, out_hbm.at[pl.ds(nbase * d, NPT * d)])

    return k(xw, row, col, nrm)


_BR = 2000


def _tc_matmul(x, wt):
    m, kdim = x.shape
    dout = wt.shape[1]

    def body(x_ref, w_ref, o_ref):
        o_ref[...] = jnp.dot(x_ref[...], w_ref[...],
                             preferred_element_type=jnp.float32)

    return pl.pallas_call(
        body,
        grid=(m // _BR,),
        in_specs=[
            pl.BlockSpec((_BR, kdim), lambda i: (i, 0)),
            pl.BlockSpec((kdim, dout), lambda i: (0, 0)),
        ],
        out_specs=pl.BlockSpec((_BR, dout), lambda i: (i, 0)),
        out_shape=jax.ShapeDtypeStruct((m, dout), jnp.float32),
    )(x, wt)


def _tc_dinv(partials):
    def body(p_ref, dinv_ref, dinv2_ref):
        deg = jnp.sum(p_ref[...], axis=0) + 1.0
        dv = jnp.where(deg > 0, lax.rsqrt(deg), 0.0)
        dinv_ref[...] = dv
        dinv2_ref[...] = (dv * dv)[:, None]

    return pl.pallas_call(
        body,
        out_shape=(
            jax.ShapeDtypeStruct((N,), jnp.float32),
            jax.ShapeDtypeStruct((N, 1), jnp.float32),
        ),
    )(partials)


def _tc_pre_bn(agg, xw, dinv2, b):
    m, d = agg.shape

    def body(agg_ref, xw_ref, d2_ref, b_ref, a_ref, st_ref):
        i = pl.program_id(0)
        a = agg_ref[...] + d2_ref[...] * xw_ref[...] + b_ref[...][None, :]
        a_ref[...] = a
        s = jnp.concatenate(
            [jnp.sum(a, axis=0, keepdims=True),
             jnp.sum(a * a, axis=0, keepdims=True)], axis=0)

        @pl.when(i == 0)
        def _():
            st_ref[...] = jnp.zeros_like(st_ref)

        st_ref[...] += s

    return pl.pallas_call(
        body,
        grid=(m // _BR,),
        in_specs=[
            pl.BlockSpec((_BR, d), lambda i: (i, 0)),
            pl.BlockSpec((_BR, d), lambda i: (i, 0)),
            pl.BlockSpec((_BR, 1), lambda i: (i, 0)),
            pl.BlockSpec((d,), lambda i: (0,)),
        ],
        out_specs=[
            pl.BlockSpec((_BR, d), lambda i: (i, 0)),
            pl.BlockSpec((2, d), lambda i: (0, 0)),
        ],
        out_shape=[
            jax.ShapeDtypeStruct((m, d), jnp.float32),
            jax.ShapeDtypeStruct((2, d), jnp.float32),
        ],
    )(agg, xw, dinv2, b)


def _tc_bn_relu_matmul(a, st, gamma, beta, wt, bias):
    m, d = a.shape
    dout = wt.shape[1]
    has_bias = bias is not None

    def body(*refs):
        if has_bias:
            a_ref, st_ref, g_ref, be_ref, w_ref, bias_ref, o_ref = refs
        else:
            a_ref, st_ref, g_ref, be_ref, w_ref, o_ref = refs
        mean = st_ref[0, :] * (1.0 / m)
        ex2 = st_ref[1, :] * (1.0 / m)
        var = ex2 - mean * mean
        inv = lax.rsqrt(var + 1e-5)
        h = (a_ref[...] - mean[None, :]) * inv[None, :] * g_ref[...][None, :] \
            + be_ref[...][None, :]
        h = jnp.maximum(h, 0.0)
        r = jnp.dot(h, w_ref[...], preferred_element_type=jnp.float32)
        if has_bias:
            r = r + bias_ref[...][None, :]
        o_ref[...] = r

    in_specs = [
        pl.BlockSpec((_BR, d), lambda i: (i, 0)),
        pl.BlockSpec((2, d), lambda i: (0, 0)),
        pl.BlockSpec((d,), lambda i: (0,)),
        pl.BlockSpec((d,), lambda i: (0,)),
        pl.BlockSpec((d, dout), lambda i: (0, 0)),
    ]
    args = [a, st, gamma, beta, wt]
    if has_bias:
        in_specs.append(pl.BlockSpec((dout,), lambda i: (0,)))
        args.append(bias)

    return pl.pallas_call(
        body,
        grid=(m // _BR,),
        in_specs=in_specs,
        out_specs=pl.BlockSpec((_BR, dout), lambda i: (i, 0)),
        out_shape=jax.ShapeDtypeStruct((m, dout), jnp.float32),
    )(*args)


def kernel(x, edge_index, edge_weight, W1, b1, g1, beta1, W2, b2, g2, beta2,
           Wl, bl):
    row = edge_index[0].astype(jnp.int32)
    col = edge_index[1].astype(jnp.int32)
    ew = edge_weight.astype(jnp.float32)
    pad = EP - E
    rowp = jnp.pad(row, (0, pad))
    colp = jnp.pad(col, (0, pad))
    ewp = jnp.pad(ew, (0, pad))

    partials = _deg_partials(colp, ewp).reshape(NW, N)
    dinv, dinv2 = _tc_dinv(partials)
    nrm = _norm_kernel(rowp, colp, ewp, dinv)

    xw1 = _tc_matmul(x, W1.T)
    agg1 = _agg_kernel(xw1, rowp, colp, nrm, 256, 256, 64).reshape(NPAD, 256)[:N]
    a1, st1 = _tc_pre_bn(agg1, xw1, dinv2, b1)
    xw2 = _tc_bn_relu_matmul(a1, st1, g1, beta1, W2.T, None)

    xw2p = jnp.pad(xw2, ((0, 0), (0, 64)))
    agg2 = _agg_kernel(xw2p, rowp, colp, nrm, 64, 128, 64).reshape(NPAD, 64)[:N]
    a2, st2 = _tc_pre_bn(agg2, xw2, dinv2, b2)
    out = _tc_bn_relu_matmul(a2, st2, g2, beta2, Wl.T, bl)
    return out


# speedup vs baseline: 1.4117x; 1.4117x over previous
import dataclasses
import functools

import jax
import jax.numpy as jnp
from jax import lax
from jax.experimental import pallas as pl
from jax.experimental.pallas import tpu as pltpu
from jax.experimental.pallas import tpu_sc as plsc

N = 10000
E = 160000
EP = 160256
NW = 32
EPW = EP // NW
NPT = 320
NPAD = NW * NPT
CL = 5008
CL3 = 3 * CL
NCH = EP // CL

_MESH = plsc.VectorSubcoreMesh(core_axis_name="c", subcore_axis_name="s")

_SC_PARAMS = pltpu.CompilerParams()
if "needs_layout_passes" in pltpu.CompilerParams.__dataclass_fields__:
    _SC_PARAMS = dataclasses.replace(_SC_PARAMS, needs_layout_passes=False)


def _wid():
    return lax.axis_index("s") * 2 + lax.axis_index("c")


def _deg_partials(col, ew):
    @functools.partial(
        pl.kernel,
        out_type=jax.ShapeDtypeStruct((NW * N,), jnp.float32),
        mesh=_MESH,
        compiler_params=_SC_PARAMS,
        scratch_types=[
            pltpu.VMEM((EPW + 16,), jnp.int32),
            pltpu.VMEM((EPW + 16,), jnp.float32),
            pltpu.VMEM((N,), jnp.float32),
        ],
    )
    def k(col_hbm, ew_hbm, out_hbm, colv, eww, hist):
        w = _wid()
        base = w * EPW
        pltpu.sync_copy(col_hbm.at[pl.ds(base, EPW)], colv.at[pl.ds(0, EPW)])
        pltpu.sync_copy(ew_hbm.at[pl.ds(base, EPW)], eww.at[pl.ds(0, EPW)])

        @pl.loop(0, N // 16)
        def _(i):
            hist[pl.ds(i * 16, 16)] = jnp.zeros((16,), jnp.float32)

        lane0 = lax.iota(jnp.int32, 16) == 0

        @pl.loop(0, EPW)
        def _(e):
            plsc.addupdate_scatter(hist, [colv[pl.ds(e, 16)]],
                                   eww[pl.ds(e, 16)], mask=lane0)

        pltpu.sync_copy(hist, out_hbm.at[pl.ds(w * N, N)])

    return k(col, ew)


def _norm_kernel(row, col, ew, dinv):
    @functools.partial(
        pl.kernel,
        out_type=jax.ShapeDtypeStruct((EP,), jnp.float32),
        mesh=_MESH,
        compiler_params=_SC_PARAMS,
        scratch_types=[
            pltpu.VMEM((EPW,), jnp.int32),
            pltpu.VMEM((EPW,), jnp.int32),
            pltpu.VMEM((EPW,), jnp.float32),
            pltpu.VMEM((EPW,), jnp.float32),
            pltpu.VMEM((N,), jnp.float32),
        ],
    )
    def k(row_hbm, col_hbm, ew_hbm, dinv_hbm, out_hbm, rv, cv, wv, nv, dv):
        w = _wid()
        base = w * EPW
        pltpu.sync_copy(row_hbm.at[pl.ds(base, EPW)], rv)
        pltpu.sync_copy(col_hbm.at[pl.ds(base, EPW)], cv)
        pltpu.sync_copy(ew_hbm.at[pl.ds(base, EPW)], wv)
        pltpu.sync_copy(dinv_hbm, dv)

        @pl.loop(0, EPW // 16)
        def _(g):
            sl = pl.ds(g * 16, 16)
            dr = plsc.load_gather(dv, [rv[sl]])
            dc = plsc.load_gather(dv, [cv[sl]])
            nv[sl] = dr * wv[sl] * dc

        pltpu.sync_copy(nv, out_hbm.at[pl.ds(base, EPW)])

    return k(row, col, ew, dinv)


def _agg_kernel(xw, row, col, nrm, d, d_tab, g_rows):
    @functools.partial(
        pl.kernel,
        out_type=jax.ShapeDtypeStruct((NPAD * d,), jnp.float32),
        mesh=_MESH,
        compiler_params=_SC_PARAMS,
        scratch_types=[
            pltpu.VMEM((CL,), jnp.int32),
            pltpu.VMEM((CL,), jnp.int32),
            pltpu.VMEM((CL,), jnp.float32),
            pltpu.VMEM((CL,), jnp.int32),
            pltpu.VMEM((CL + 16,), jnp.int32),
            pltpu.VMEM((CL + 16,), jnp.float32),
            pltpu.VMEM((g_rows, d_tab), jnp.float32),
            pltpu.VMEM((NPT * d,), jnp.float32),
        ],
    )
    def k(xw_hbm, row_hbm, col_hbm, nrm_hbm, out_hbm, rs, cs, ns, rb, lb, nb, gb, acc):
        w = _wid()
        nbase = w * NPT

        @pl.loop(0, NPT * d // 16)
        def _(i):
            acc[pl.ds(i * 16, 16)] = jnp.zeros((16,), jnp.float32)

        @pl.loop(0, CL // 16)
        def _(i):
            rb[pl.ds(i * 16, 16)] = jnp.zeros((16,), jnp.int32)

        @pl.loop(0, NCH)
        def _(ch):
            eoff = ch * CL
            pltpu.sync_copy(row_hbm.at[pl.ds(eoff, CL)], rs)
            pltpu.sync_copy(col_hbm.at[pl.ds(eoff, CL)], cs)
            pltpu.sync_copy(nrm_hbm.at[pl.ds(eoff, CL)], ns)

            def grp(g, pos):
                sl = pl.ds(g * 16, 16)
                r16 = rs[sl]
                c16 = cs[sl]
                n16 = ns[sl]
                lc = c16 - nbase
                m = (lc >= 0) & (lc < NPT) & (c16 < N)
                plsc.store_compressed(rb.at[pl.ds(pos, 16)], r16, mask=m)
                plsc.store_compressed(lb.at[pl.ds(pos, 16)], lc, mask=m)
                plsc.store_compressed(nb.at[pl.ds(pos, 16)], n16, mask=m)
                return pos + jnp.sum(m.astype(jnp.int32))

            n_in = lax.fori_loop(0, CL // 16, grp, 0)
            nblk = (n_in + g_rows - 1) // g_rows

            def blk(b, _):
                pltpu.sync_copy(xw_hbm.at[rb.at[pl.ds(b * g_rows, g_rows)]],
                                gb)
                nleft = jnp.minimum(n_in - b * g_rows, g_rows)

                def edge(e, _):
                    pos = b * g_rows + e
                    lc_s = lb[pl.ds(pos, 16)][0]
                    nmv = jnp.full((16,), nb[pl.ds(pos, 16)][0], jnp.float32)
                    for f in range(d // 16):
                        g16 = gb[e, pl.ds(f * 16, 16)]
                        plsc.addupdate(acc.at[pl.ds(lc_s * d + f * 16, 16)],
                                       nmv * g16)
                    return 0

                lax.fori_loop(0, nleft, edge, 0)
                return 0

            lax.fori_loop(0, nblk, blk, 0)

        pltpu.sync_copy(acc, out_hbm.at[pl.ds(nbase * d, NPT * d)])

    return k(xw, row, col, nrm)


_BR = 2000


def _tc_matmul(x, wt):
    m, kdim = x.shape
    dout = wt.shape[1]

    def body(x_ref, w_ref, o_ref):
        o_ref[...] = jnp.dot(x_ref[...], w_ref[...],
                             preferred_element_type=jnp.float32)

    return pl.pallas_call(
        body,
        grid=(m // _BR,),
        in_specs=[
            pl.BlockSpec((_BR, kdim), lambda i: (i, 0)),
            pl.BlockSpec((kdim, dout), lambda i: (0, 0)),
        ],
        out_specs=pl.BlockSpec((_BR, dout), lambda i: (i, 0)),
        out_shape=jax.ShapeDtypeStruct((m, dout), jnp.float32),
    )(x, wt)


def _tc_dinv(partials):
    def body(p_ref, dinv_ref, dinv2_ref):
        deg = jnp.sum(p_ref[...], axis=0) + 1.0
        dv = jnp.where(deg > 0, lax.rsqrt(deg), 0.0)
        dinv_ref[...] = dv
        dinv2_ref[...] = (dv * dv)[:, None]

    return pl.pallas_call(
        body,
        out_shape=(
            jax.ShapeDtypeStruct((N,), jnp.float32),
            jax.ShapeDtypeStruct((N, 1), jnp.float32),
        ),
    )(partials)


def _tc_pre_bn(agg, xw, dinv2, b):
    m, d = agg.shape

    def body(agg_ref, xw_ref, d2_ref, b_ref, a_ref, st_ref):
        i = pl.program_id(0)
        a = agg_ref[...] + d2_ref[...] * xw_ref[...] + b_ref[...][None, :]
        a_ref[...] = a
        s = jnp.concatenate(
            [jnp.sum(a, axis=0, keepdims=True),
             jnp.sum(a * a, axis=0, keepdims=True)], axis=0)

        @pl.when(i == 0)
        def _():
            st_ref[...] = jnp.zeros_like(st_ref)

        st_ref[...] += s

    return pl.pallas_call(
        body,
        grid=(m // _BR,),
        in_specs=[
            pl.BlockSpec((_BR, d), lambda i: (i, 0)),
            pl.BlockSpec((_BR, d), lambda i: (i, 0)),
            pl.BlockSpec((_BR, 1), lambda i: (i, 0)),
            pl.BlockSpec((d,), lambda i: (0,)),
        ],
        out_specs=[
            pl.BlockSpec((_BR, d), lambda i: (i, 0)),
            pl.BlockSpec((2, d), lambda i: (0, 0)),
        ],
        out_shape=[
            jax.ShapeDtypeStruct((m, d), jnp.float32),
            jax.ShapeDtypeStruct((2, d), jnp.float32),
        ],
    )(agg, xw, dinv2, b)


def _tc_bn_relu_matmul(a, st, gamma, beta, wt, bias):
    m, d = a.shape
    dout = wt.shape[1]
    has_bias = bias is not None

    def body(*refs):
        if has_bias:
            a_ref, st_ref, g_ref, be_ref, w_ref, bias_ref, o_ref = refs
        else:
            a_ref, st_ref, g_ref, be_ref, w_ref, o_ref = refs
        mean = st_ref[0, :] * (1.0 / m)
        ex2 = st_ref[1, :] * (1.0 / m)
        var = ex2 - mean * mean
        inv = lax.rsqrt(var + 1e-5)
        h = (a_ref[...] - mean[None, :]) * inv[None, :] * g_ref[...][None, :] \
            + be_ref[...][None, :]
        h = jnp.maximum(h, 0.0)
        r = jnp.dot(h, w_ref[...], preferred_element_type=jnp.float32)
        if has_bias:
            r = r + bias_ref[...][None, :]
        o_ref[...] = r

    in_specs = [
        pl.BlockSpec((_BR, d), lambda i: (i, 0)),
        pl.BlockSpec((2, d), lambda i: (0, 0)),
        pl.BlockSpec((d,), lambda i: (0,)),
        pl.BlockSpec((d,), lambda i: (0,)),
        pl.BlockSpec((d, dout), lambda i: (0, 0)),
    ]
    args = [a, st, gamma, beta, wt]
    if has_bias:
        in_specs.append(pl.BlockSpec((dout,), lambda i: (0,)))
        args.append(bias)

    return pl.pallas_call(
        body,
        grid=(m // _BR,),
        in_specs=in_specs,
        out_specs=pl.BlockSpec((_BR, dout), lambda i: (i, 0)),
        out_shape=jax.ShapeDtypeStruct((m, dout), jnp.float32),
    )(*args)


def kernel(x, edge_index, edge_weight, W1, b1, g1, beta1, W2, b2, g2, beta2,
           Wl, bl):
    row = edge_index[0].astype(jnp.int32)
    col = edge_index[1].astype(jnp.int32)
    ew = edge_weight.astype(jnp.float32)
    pad = EP - E
    rowp = jnp.pad(row, (0, pad))
    colp = jnp.pad(col, (0, pad))
    ewp = jnp.pad(ew, (0, pad))

    partials = _deg_partials(colp, ewp).reshape(NW, N)
    dinv, dinv2 = _tc_dinv(partials)
    nrm = _norm_kernel(rowp, colp, ewp, dinv)

    xw1 = _tc_matmul(x, W1.T)
    agg1 = _agg_kernel(xw1, rowp, colp, nrm, 256, 256, 64).reshape(NPAD, 256)[:N]
    a1, st1 = _tc_pre_bn(agg1, xw1, dinv2, b1)
    xw2 = _tc_bn_relu_matmul(a1, st1, g1, beta1, W2.T, None)

    xw2p = jnp.pad(xw2, ((0, 0), (0, 64)))
    agg2 = _agg_kernel(xw2p, rowp, colp, nrm, 64, 128, 64).reshape(NPAD, 64)[:N]
    a2, st2 = _tc_pre_bn(agg2, xw2, dinv2, b2)
    out = _tc_bn_relu_matmul(a2, st2, g2, beta2, Wl.T, bl)
    return out
